# baseline (speedup 1.0000x reference)
import dataclasses
import functools

import jax
import jax.numpy as jnp
from jax.experimental import pallas as pl
from jax.experimental.pallas import tpu as pltpu
from jax.experimental.pallas import tpu_sc as plsc

_E = 8
_D = 4

_ROWS = 4096
_COLS = 4096
_BLOCK_ROWS = 32
_CHUNK = 128


def _vq_tc_kernel(cb_ref, x_ref, o_ref):
    lane = jax.lax.broadcasted_iota(jnp.int32, (1, _CHUNK), 1)
    d = lane & 3
    md0 = d == 0
    md1 = d == 1
    md2 = d == 2

    for c in range(0, _COLS, _CHUNK):
        xr = x_ref[:, c:c + _CHUNK]
        x = xr.astype(jnp.bfloat16).astype(jnp.float32)

        r = {s: jnp.roll(x, s, axis=1) for s in (-3, -2, -1, 1, 2, 3)}
        y0 = jnp.where(md0, x, jnp.where(md1, r[1], jnp.where(md2, r[2], r[3])))
        y1 = jnp.where(md0, r[-1], jnp.where(md1, x, jnp.where(md2, r[1], r[2])))
        y2 = jnp.where(md0, r[-2], jnp.where(md1, r[-1], jnp.where(md2, x, r[1])))
        y3 = jnp.where(md0, r[-3], jnp.where(md1, r[-2], jnp.where(md2, r[-1], x)))

        best = None
        q = None
        for j in range(_E):
            c0 = cb_ref[j, 0]
            c1 = cb_ref[j, 1]
            c2 = cb_ref[j, 2]
            c3 = cb_ref[j, 3]
            c2sum = cb_ref[j, 4]
            dot = ((y0 * c0 + y1 * c1) + y2 * c2) + y3 * c3
            s = c2sum - (dot + dot)
            cj = jnp.where(md0, c0, jnp.where(md1, c1, jnp.where(md2, c2, c3)))
            cj = jnp.broadcast_to(cj, x.shape)
            if best is None:
                best, q = s, cj
            else:
                m = s < best
                best = jnp.minimum(best, s)
                q = jnp.where(m, cj, q)
        o_ref[:, c:c + _CHUNK] = q


def _bf16_round(x):
    u = plsc.bitcast(x, jnp.int32)
    r = u + (jnp.int32(0x7FFF) + ((u >> 16) & 1))
    return plsc.bitcast(r & jnp.int32(-65536), jnp.float32)


def _vq_sc_body(cb_vm, x_vm, o_vm, ncols):
    iota = jax.lax.iota(jnp.int32, 16)
    idx4 = iota * 4
    zeros = jnp.zeros((16,), jnp.int32)
    dsplat = [jnp.full((16,), d, jnp.int32) for d in range(4)]
    cs = [[plsc.load_gather(cb_vm, [jnp.full((16,), j + 1, jnp.int32), dsplat[m]])
           for m in range(4)] for j in range(_E)]
    c2s = [plsc.load_gather(cb_vm, [jnp.full((16,), j + 1, jnp.int32),
                                    jnp.full((16,), 4, jnp.int32)])
           for j in range(_E)]

    @pl.loop(0, ncols, step=64)
    def _(c):
        i0 = idx4 + c
        idxs = [i0, i0 + 1, i0 + 2, i0 + 3]
        xb = [_bf16_round(plsc.load_gather(x_vm, [zeros, i])) for i in idxs]
        best = None
        bj = None
        for j in range(_E):
            dot = ((xb[0] * cs[j][0] + xb[1] * cs[j][1])
                   + xb[2] * cs[j][2]) + xb[3] * cs[j][3]
            s = c2s[j] - (dot + dot)
            if best is None:
                best, bj = s, jnp.full((16,), 1, jnp.int32)
            else:
                m = s < best
                best = jnp.minimum(best, s)
                bj = jnp.where(m, j + 1, bj)
        for d in range(4):
            qd = plsc.load_gather(cb_vm, [bj, dsplat[d]])
            plsc.store_scatter(o_vm, [zeros, idxs[d]], qd)


def _vq_sc(cb_aug, xf, rows):
    mesh = plsc.VectorSubcoreMesh(core_axis_name="c", subcore_axis_name="s")
    cp = pltpu.CompilerParams()
    if "needs_layout_passes" in pltpu.CompilerParams.__dataclass_fields__:
        cp = dataclasses.replace(cp, needs_layout_passes=False)

    @functools.partial(
        pl.kernel,
        out_type=jax.ShapeDtypeStruct((rows, _COLS), jnp.float32),
        mesh=mesh,
        compiler_params=cp,
        scratch_types=[
            pltpu.VMEM((_E + 1, 5), jnp.float32),
        ],
    )
    def k(cb_hbm, x_hbm, o_hbm, cb_vm):
        pltpu.sync_copy(cb_hbm, cb_vm)

        def body(x_vm, o_vm):
            _vq_sc_body(cb_vm, x_vm, o_vm, _COLS)

        pltpu.emit_pipeline(
            body,
            grid=(rows,),
            in_specs=[pl.BlockSpec((1, _COLS), lambda i: (i, 0))],
            out_specs=[pl.BlockSpec((1, _COLS), lambda i: (i, 0))],
            core_axis_name=("c", "s"),
            dimension_semantics=(pltpu.PARALLEL,),
        )(x_hbm, o_hbm)

    return k(cb_aug, xf)


@jax.jit
def kernel(latent, codebook):
    xf = latent.reshape(_ROWS, _COLS)
    c2 = jnp.sum(codebook ** 2, axis=-1)
    cb_r = codebook.astype(jnp.bfloat16).astype(jnp.float32)
    cb_aug = jnp.concatenate([cb_r, c2[:, None]], axis=1)
    cb_aug = jnp.concatenate([jnp.zeros((1, 5), jnp.float32), cb_aug], axis=0)

    q = _vq_sc(cb_aug, xf, _ROWS)
    qr = q.reshape(latent.shape)
    return (qr, qr)


def _vq_tc(cb_aug, xf):
    grid = (_ROWS // _BLOCK_ROWS,)
    return pl.pallas_call(
        _vq_tc_kernel,
        grid=grid,
        in_specs=[
            pl.BlockSpec(memory_space=pltpu.SMEM),
            pl.BlockSpec((_BLOCK_ROWS, _COLS), lambda i: (i, 0)),
        ],
        out_specs=pl.BlockSpec((_BLOCK_ROWS, _COLS), lambda i: (i, 0)),
        out_shape=jax.ShapeDtypeStruct((_ROWS, _COLS), jnp.float32),
        compiler_params=pltpu.CompilerParams(
            dimension_semantics=("arbitrary",),
        ),
    )(cb_aug, xf)


# speedup vs baseline: 1.1954x; 1.1954x over previous
import dataclasses
import functools

import jax
import jax.numpy as jnp
from jax.experimental import pallas as pl
from jax.experimental.pallas import tpu as pltpu
from jax.experimental.pallas import tpu_sc as plsc

_E = 8
_D = 4

_ROWS = 4096
_COLS = 4096
_BLOCK_ROWS = 32
_CHUNK = 128
_SC_BLOCK = 2


def _vq_tc_kernel(cb_ref, x_ref, o_ref):
    lane = jax.lax.broadcasted_iota(jnp.int32, (1, _CHUNK), 1)
    d = lane & 3
    md0 = d == 0
    md1 = d == 1
    md2 = d == 2

    for c in range(0, _COLS, _CHUNK):
        xr = x_ref[:, c:c + _CHUNK]
        x = xr.astype(jnp.bfloat16).astype(jnp.float32)

        r = {s: jnp.roll(x, s, axis=1) for s in (-3, -2, -1, 1, 2, 3)}
        y0 = jnp.where(md0, x, jnp.where(md1, r[1], jnp.where(md2, r[2], r[3])))
        y1 = jnp.where(md0, r[-1], jnp.where(md1, x, jnp.where(md2, r[1], r[2])))
        y2 = jnp.where(md0, r[-2], jnp.where(md1, r[-1], jnp.where(md2, x, r[1])))
        y3 = jnp.where(md0, r[-3], jnp.where(md1, r[-2], jnp.where(md2, r[-1], x)))

        best = None
        q = None
        for j in range(_E):
            c0 = cb_ref[j, 0]
            c1 = cb_ref[j, 1]
            c2 = cb_ref[j, 2]
            c3 = cb_ref[j, 3]
            c2sum = cb_ref[j, 4]
            dot = ((y0 * c0 + y1 * c1) + y2 * c2) + y3 * c3
            s = c2sum - (dot + dot)
            cj = jnp.where(md0, c0, jnp.where(md1, c1, jnp.where(md2, c2, c3)))
            cj = jnp.broadcast_to(cj, x.shape)
            if best is None:
                best, q = s, cj
            else:
                m = s < best
                best = jnp.minimum(best, s)
                q = jnp.where(m, cj, q)
        o_ref[:, c:c + _CHUNK] = q


def _bf16_round(x):
    u = plsc.bitcast(x, jnp.int32)
    r = u + (jnp.int32(0x7FFF) + ((u >> 16) & 1))
    return plsc.bitcast(r & jnp.int32(-65536), jnp.float32)


def _vq_sc_body(cb_vm, x_vm, o_vm):
    nrows, ncols = x_vm.shape
    iota = jax.lax.iota(jnp.int32, 16)
    idx4 = iota * 4
    zeros = jnp.zeros((16,), jnp.int32)
    dsplat = [jnp.full((16,), d, jnp.int32) for d in range(4)]
    cs = [[plsc.load_gather(cb_vm, [jnp.full((16,), j + 1, jnp.int32), dsplat[m]])
           for m in range(4)] for j in range(_E)]
    c2s = [plsc.load_gather(cb_vm, [jnp.full((16,), j + 1, jnp.int32),
                                    jnp.full((16,), 4, jnp.int32)])
           for j in range(_E)]

    for rr in range(nrows):
        rsplat = zeros if rr == 0 else jnp.full((16,), rr, jnp.int32)

        @plsc.parallel_loop(0, ncols, step=64, unroll=4)
        def _(c, rsplat=rsplat):
            i0 = idx4 + c
            idxs = [i0, i0 + 1, i0 + 2, i0 + 3]
            xb = [_bf16_round(plsc.load_gather(x_vm, [rsplat, i])) for i in idxs]
            best = None
            bj = None
            for j in range(_E):
                dot = ((xb[0] * cs[j][0] + xb[1] * cs[j][1])
                       + xb[2] * cs[j][2]) + xb[3] * cs[j][3]
                s = c2s[j] - (dot + dot)
                if best is None:
                    best, bj = s, jnp.full((16,), 1, jnp.int32)
                else:
                    m = s < best
                    best = jnp.minimum(best, s)
                    bj = jnp.where(m, j + 1, bj)
            for d in range(4):
                qd = plsc.load_gather(cb_vm, [bj, dsplat[d]])
                plsc.store_scatter(o_vm, [rsplat, idxs[d]], qd)


def _vq_sc(cb_aug, xf, rows):
    mesh = plsc.VectorSubcoreMesh(core_axis_name="c", subcore_axis_name="s")
    cp = pltpu.CompilerParams()
    if "needs_layout_passes" in pltpu.CompilerParams.__dataclass_fields__:
        cp = dataclasses.replace(cp, needs_layout_passes=False)

    @functools.partial(
        pl.kernel,
        out_type=jax.ShapeDtypeStruct((rows, _COLS), jnp.float32),
        mesh=mesh,
        compiler_params=cp,
        scratch_types=[
            pltpu.VMEM((_E + 1, 5), jnp.float32),
        ],
    )
    def k(cb_hbm, x_hbm, o_hbm, cb_vm):
        pltpu.sync_copy(cb_hbm, cb_vm)

        def body(x_vm, o_vm):
            _vq_sc_body(cb_vm, x_vm, o_vm)

        pltpu.emit_pipeline(
            body,
            grid=(rows // _SC_BLOCK,),
            in_specs=[pl.BlockSpec((_SC_BLOCK, _COLS), lambda i: (i, 0))],
            out_specs=[pl.BlockSpec((_SC_BLOCK, _COLS), lambda i: (i, 0))],
            core_axis_name=("c", "s"),
            dimension_semantics=(pltpu.PARALLEL,),
        )(x_hbm, o_hbm)

    return k(cb_aug, xf)


@jax.jit
def kernel(latent, codebook):
    xf = latent.reshape(_ROWS, _COLS)
    c2 = jnp.sum(codebook ** 2, axis=-1)
    cb_r = codebook.astype(jnp.bfloat16).astype(jnp.float32)
    cb_aug = jnp.concatenate([cb_r, c2[:, None]], axis=1)
    cb_aug = jnp.concatenate([jnp.zeros((1, 5), jnp.float32), cb_aug], axis=0)

    q = _vq_sc(cb_aug, xf, _ROWS)
    qr = q.reshape(latent.shape)
    return (qr, qr)


def _vq_tc(cb_aug, xf):
    grid = (_ROWS // _BLOCK_ROWS,)
    return pl.pallas_call(
        _vq_tc_kernel,
        grid=grid,
        in_specs=[
            pl.BlockSpec(memory_space=pltpu.SMEM),
            pl.BlockSpec((_BLOCK_ROWS, _COLS), lambda i: (i, 0)),
        ],
        out_specs=pl.BlockSpec((_BLOCK_ROWS, _COLS), lambda i: (i, 0)),
        out_shape=jax.ShapeDtypeStruct((_ROWS, _COLS), jnp.float32),
        compiler_params=pltpu.CompilerParams(
            dimension_semantics=("arbitrary",),
        ),
    )(cb_aug, xf)


# speedup vs baseline: 1.5564x; 1.3020x over previous
import dataclasses
import functools

import jax
import jax.numpy as jnp
from jax.experimental import pallas as pl
from jax.experimental.pallas import tpu as pltpu
from jax.experimental.pallas import tpu_sc as plsc

_E = 8
_D = 4

_ROWS = 4096
_COLS = 4096
_BLOCK_ROWS = 32
_CHUNK = 128
_SC_BLOCK = 2
_SC_ROWS = 1792


def _vq_tc_kernel(cb_ref, x_ref, o_ref):
    lane = jax.lax.broadcasted_iota(jnp.int32, (1, _CHUNK), 1)
    d = lane & 3
    md0 = d == 0
    md1 = d == 1
    md2 = d == 2

    for c in range(0, _COLS, _CHUNK):
        xr = x_ref[:, c:c + _CHUNK]
        x = xr.astype(jnp.bfloat16).astype(jnp.float32)

        r = {s: jnp.roll(x, s, axis=1) for s in (-3, -2, -1, 1, 2, 3)}
        y0 = jnp.where(md0, x, jnp.where(md1, r[1], jnp.where(md2, r[2], r[3])))
        y1 = jnp.where(md0, r[-1], jnp.where(md1, x, jnp.where(md2, r[1], r[2])))
        y2 = jnp.where(md0, r[-2], jnp.where(md1, r[-1], jnp.where(md2, x, r[1])))
        y3 = jnp.where(md0, r[-3], jnp.where(md1, r[-2], jnp.where(md2, r[-1], x)))

        best = None
        q = None
        for j in range(_E):
            c0 = cb_ref[j, 0]
            c1 = cb_ref[j, 1]
            c2 = cb_ref[j, 2]
            c3 = cb_ref[j, 3]
            c2sum = cb_ref[j, 4]
            dot = ((y0 * c0 + y1 * c1) + y2 * c2) + y3 * c3
            s = c2sum - (dot + dot)
            cj = jnp.where(md0, c0, jnp.where(md1, c1, jnp.where(md2, c2, c3)))
            cj = jnp.broadcast_to(cj, x.shape)
            if best is None:
                best, q = s, cj
            else:
                m = s < best
                best = jnp.minimum(best, s)
                q = jnp.where(m, cj, q)
        o_ref[:, c:c + _CHUNK] = q


def _bf16_round(x):
    u = plsc.bitcast(x, jnp.int32)
    r = u + (jnp.int32(0x7FFF) + ((u >> 16) & 1))
    return plsc.bitcast(r & jnp.int32(-65536), jnp.float32)


def _vq_sc_body(cb_vm, x_vm, o_vm):
    nrows, ncols = x_vm.shape
    iota = jax.lax.iota(jnp.int32, 16)
    idx4 = iota * 4
    zeros = jnp.zeros((16,), jnp.int32)
    dsplat = [jnp.full((16,), d, jnp.int32) for d in range(4)]
    cs = [[plsc.load_gather(cb_vm, [jnp.full((16,), j + 1, jnp.int32), dsplat[m]])
           for m in range(4)] for j in range(_E)]
    c2s = [plsc.load_gather(cb_vm, [jnp.full((16,), j + 1, jnp.int32),
                                    jnp.full((16,), 4, jnp.int32)])
           for j in range(_E)]

    for rr in range(nrows):
        rsplat = zeros if rr == 0 else jnp.full((16,), rr, jnp.int32)

        @plsc.parallel_loop(0, ncols, step=64, unroll=4)
        def _(c, rsplat=rsplat):
            i0 = idx4 + c
            idxs = [i0, i0 + 1, i0 + 2, i0 + 3]
            xb = [_bf16_round(plsc.load_gather(x_vm, [rsplat, i])) for i in idxs]
            best = None
            bj = None
            for j in range(_E):
                dot = ((xb[0] * cs[j][0] + xb[1] * cs[j][1])
                       + xb[2] * cs[j][2]) + xb[3] * cs[j][3]
                s = c2s[j] - (dot + dot)
                if best is None:
                    best, bj = s, jnp.full((16,), 1, jnp.int32)
                else:
                    m = s < best
                    best = jnp.minimum(best, s)
                    bj = jnp.where(m, j + 1, bj)
            for d in range(4):
                qd = plsc.load_gather(cb_vm, [bj, dsplat[d]])
                plsc.store_scatter(o_vm, [rsplat, idxs[d]], qd)


def _vq_sc(cb_aug, xf, rows):
    mesh = plsc.VectorSubcoreMesh(core_axis_name="c", subcore_axis_name="s")
    cp = pltpu.CompilerParams()
    if "needs_layout_passes" in pltpu.CompilerParams.__dataclass_fields__:
        cp = dataclasses.replace(cp, needs_layout_passes=False)

    @functools.partial(
        pl.kernel,
        out_type=jax.ShapeDtypeStruct((rows, _COLS), jnp.float32),
        mesh=mesh,
        compiler_params=cp,
        scratch_types=[
            pltpu.VMEM((_E + 1, 5), jnp.float32),
        ],
    )
    def k(cb_hbm, x_hbm, o_hbm, cb_vm):
        pltpu.sync_copy(cb_hbm, cb_vm)

        def body(x_vm, o_vm):
            _vq_sc_body(cb_vm, x_vm, o_vm)

        pltpu.emit_pipeline(
            body,
            grid=(rows // _SC_BLOCK,),
            in_specs=[pl.BlockSpec((_SC_BLOCK, _COLS), lambda i: (i, 0))],
            out_specs=[pl.BlockSpec((_SC_BLOCK, _COLS), lambda i: (i, 0))],
            core_axis_name=("c", "s"),
            dimension_semantics=(pltpu.PARALLEL,),
        )(x_hbm, o_hbm)

    return k(cb_aug, xf)


@jax.jit
def kernel(latent, codebook):
    xf = latent.reshape(_ROWS, _COLS)
    c2 = jnp.sum(codebook ** 2, axis=-1)
    cb_r = codebook.astype(jnp.bfloat16).astype(jnp.float32)
    cb_aug = jnp.concatenate([cb_r, c2[:, None]], axis=1)
    cb_aug9 = jnp.concatenate([jnp.zeros((1, 5), jnp.float32), cb_aug], axis=0)

    q_sc = _vq_sc(cb_aug9, xf[:_SC_ROWS], _SC_ROWS)
    q_tc = _vq_tc(cb_aug, xf[_SC_ROWS:])
    q = jnp.concatenate([q_sc, q_tc], axis=0)
    qr = q.reshape(latent.shape)
    return (qr, qr)


def _vq_tc(cb_aug, xf):
    rows = xf.shape[0]
    grid = (rows // _BLOCK_ROWS,)
    return pl.pallas_call(
        _vq_tc_kernel,
        grid=grid,
        in_specs=[
            pl.BlockSpec(memory_space=pltpu.SMEM),
            pl.BlockSpec((_BLOCK_ROWS, _COLS), lambda i: (i, 0)),
        ],
        out_specs=pl.BlockSpec((_BLOCK_ROWS, _COLS), lambda i: (i, 0)),
        out_shape=jax.ShapeDtypeStruct((rows, _COLS), jnp.float32),
        compiler_params=pltpu.CompilerParams(
            dimension_semantics=("arbitrary",),
        ),
    )(cb_aug, xf)
